# Initial kernel scaffold; baseline (speedup 1.0000x reference)
import jax
import jax.numpy as jnp
from jax.experimental import pallas as pl


def kernel(data, table):
    raise NotImplementedError("write your pallas kernel here")


# speedup vs baseline: 1.5621x; 1.5621x over previous
import functools

import jax
import jax.numpy as jnp
from jax import lax
from jax.experimental import pallas as pl
from jax.experimental.pallas import tpu as pltpu
from jax.experimental.pallas import tpu_sc as plsc

_FIELDS = 26
_FIELD_DIM = 38462
_EMBED = 32
_BATCH = 16384
_N = _BATCH * _FIELDS
_NC = 2
_NS = 16
_NW = _NC * _NS
_PER_W = _N // _NW
_CHUNK = 1664
_NCHUNK = _PER_W // _CHUNK
_VECS = _PER_W // 16


@jax.jit
def _sc_embed(data_flat, table):
    mesh = plsc.VectorSubcoreMesh(core_axis_name="c", subcore_axis_name="s")

    @functools.partial(
        pl.kernel,
        mesh=mesh,
        compiler_params=pltpu.CompilerParams(use_tc_tiling_on_sc=False),
        out_type=jax.ShapeDtypeStruct((_N, _EMBED), jnp.float32),
        scratch_types=[
            pltpu.VMEM((_PER_W,), jnp.int32),
            pltpu.VMEM((_CHUNK, _EMBED), jnp.float32),
            pltpu.SemaphoreType.DMA,
        ],
    )
    def k(data_hbm, table_hbm, out_hbm, idx_v, rows_v, sem):
        wid = lax.axis_index("s") * _NC + lax.axis_index("c")
        base = wid * _PER_W
        pltpu.sync_copy(data_hbm.at[pl.ds(base, _PER_W)], idx_v)

        lanes = lax.iota(jnp.int32, 16)

        def body(j, carry):
            s = j * 16
            pos = base + s + lanes
            field = lax.rem(pos, _FIELDS)
            idx_v[pl.ds(s, 16)] = idx_v[pl.ds(s, 16)] + field * _FIELD_DIM
            return carry

        lax.fori_loop(0, _VECS, body, 0)

        for c in range(_NCHUNK):
            off = c * _CHUNK
            pltpu.async_copy(
                table_hbm.at[idx_v.at[pl.ds(off, _CHUNK)]], rows_v, sem
            ).wait()
            pltpu.sync_copy(rows_v, out_hbm.at[pl.ds(base + off, _CHUNK)])

    return k(data_flat, table)


def kernel(data, table):
    out = _sc_embed(data.reshape(-1), table)
    return out.reshape(_BATCH, _FIELDS, _EMBED)


# speedup vs baseline: 3.0648x; 1.9620x over previous
import functools

import jax
import jax.numpy as jnp
from jax import lax
from jax.experimental import pallas as pl
from jax.experimental.pallas import tpu as pltpu
from jax.experimental.pallas import tpu_sc as plsc

_FIELDS = 26
_FIELD_DIM = 38462
_EMBED = 32
_BATCH = 16384
_N = _BATCH * _FIELDS
_HALF_B = _BATCH // 2
_VECS = _HALF_B // 16
_SEG_SP = 40960
_SEG_HALF = _SEG_SP // 2
_SEG_COLS = 4096
_SEG_CHUNKS = _SEG_SP // _SEG_COLS
_TBL_ALIGNED = 999936
_TAIL = 1000012 - _TBL_ALIGNED
_TAIL_CUT = _FIELD_DIM - _TAIL


def _seg(f):
    start = f * _FIELD_DIM
    a0 = start & ~127
    ln = -(-((start - a0) + _FIELD_DIM) // 128) * 128
    if a0 + ln > _TBL_ALIGNED:
        a0 -= 128
        ln = _TBL_ALIGNED - a0
    return a0, start - a0, ln


@jax.jit
def _sc_embed(data_flat, table_t, tail_t):
    mesh = plsc.VectorSubcoreMesh(core_axis_name="c", subcore_axis_name="s")

    @functools.partial(
        pl.kernel,
        mesh=mesh,
        compiler_params=pltpu.CompilerParams(
            use_tc_tiling_on_sc=True,
            needs_layout_passes=False,
            internal_scratch_in_bytes=1 << 18,
        ),
        out_type=jax.ShapeDtypeStruct((_N * _EMBED,), jnp.float32),
        scratch_types=[
            pltpu.VMEM_SHARED((16, _SEG_HALF), jnp.float32),
            pltpu.VMEM_SHARED((16, _SEG_HALF), jnp.float32),
            pltpu.VMEM_SHARED((16, 128), jnp.float32),
            pltpu.VMEM((16, _SEG_COLS), jnp.float32),
            pltpu.VMEM((1, 128), jnp.float32),
            pltpu.VMEM((_HALF_B,), jnp.int32),
            pltpu.VMEM((_HALF_B,), jnp.float32),
            pltpu.SemaphoreType.DMA,
            pltpu.SemaphoreType.DMA,
        ],
    )
    def k(data_hbm, table_hbm, tail_hbm, out_hbm, spmem_a, spmem_b,
          spmem_tail, seg_v, tail_v, idx_v, out_v, sblk, srow):
        c = lax.axis_index("c")
        s = lax.axis_index("s")
        e = c * 16 + s
        erow = pl.multiple_of(c * 16, 16)
        zeros16 = jnp.zeros((16,), jnp.int32)

        def blk_args(f):
            a0, _, ln = _seg(f)
            return [
                (
                    table_hbm.at[pl.ds(erow, 16), pl.ds(a0, _SEG_HALF)],
                    spmem_a,
                    sblk,
                ),
                (
                    table_hbm.at[
                        pl.ds(erow, 16), pl.ds(a0 + _SEG_HALF, ln - _SEG_HALF)
                    ],
                    spmem_b.at[:, pl.ds(0, ln - _SEG_HALF)],
                    sblk,
                ),
            ]

        def blk_start(f):
            @pl.when(s == 0)
            def _():
                for args in blk_args(f):
                    pltpu.async_copy(*args)

        def blk_wait(f):
            @pl.when(s == 0)
            def _():
                for args in blk_args(f):
                    pltpu.make_async_copy(*args).wait()

        def row_copy():
            dmas = []
            for r in range(_SEG_CHUNKS):
                col = r * _SEG_COLS
                sp = spmem_a if col < _SEG_HALF else spmem_b
                col = col if col < _SEG_HALF else col - _SEG_HALF
                dmas.append(
                    pltpu.async_copy(
                        sp.at[pl.ds(s, 1), pl.ds(col, _SEG_COLS)],
                        seg_v.at[pl.ds(r, 1), :],
                        srow,
                    )
                )
            for d in dmas:
                d.wait()

        @pl.when(s == 0)
        def _():
            pltpu.async_copy(tail_hbm.at[pl.ds(erow, 16), :], spmem_tail, sblk)
        blk_start(0)

        @pl.when(s == 0)
        def _():
            pltpu.make_async_copy(
                tail_hbm.at[pl.ds(erow, 16), :], spmem_tail, sblk
            ).wait()
        blk_wait(0)
        plsc.subcore_barrier()
        pltpu.sync_copy(spmem_tail.at[pl.ds(s, 1), :], tail_v)
        row_copy()
        plsc.subcore_barrier()
        blk_start(1)

        for f in range(_FIELDS):
            delta = _seg(f)[1]

            if f < _FIELDS - 1:
                def gather(j, carry):
                    iv = idx_v[pl.ds(j * 16, 16)] + delta
                    out_v[pl.ds(j * 16, 16)] = plsc.load_gather(
                        seg_v, [lax.shift_right_logical(iv, 12), iv & 4095]
                    )
                    return carry
            else:
                def gather(j, carry):
                    raw = idx_v[pl.ds(j * 16, 16)]
                    iv = jnp.minimum(raw, _TAIL_CUT - 1) + delta
                    main = plsc.load_gather(
                        seg_v, [lax.shift_right_logical(iv, 12), iv & 4095]
                    )
                    tail = plsc.load_gather(
                        tail_v, [zeros16, jnp.maximum(raw - _TAIL_CUT, 0)]
                    )
                    out_v[pl.ds(j * 16, 16)] = jnp.where(
                        raw < _TAIL_CUT, main, tail
                    )
                    return carry

            for h in range(2):
                pltpu.sync_copy(
                    data_hbm.at[pl.ds(f * _BATCH + h * _HALF_B, _HALF_B)],
                    idx_v,
                )
                lax.fori_loop(0, _VECS, gather, 0)
                dst = pl.multiple_of(
                    (f * _EMBED + e) * _BATCH + h * _HALF_B, _HALF_B
                )
                pltpu.sync_copy(out_v, out_hbm.at[pl.ds(dst, _HALF_B)])

            if f + 1 < _FIELDS:
                blk_wait(f + 1)
                plsc.subcore_barrier()
                row_copy()
                plsc.subcore_barrier()
                if f + 2 < _FIELDS:
                    blk_start(f + 2)

    return k(data_flat, table_t, tail_t)


def kernel(data, table):
    table_t = table.T
    tail_t = jnp.pad(table_t[:, _TBL_ALIGNED:], ((0, 0), (0, 128 - _TAIL)))
    out = _sc_embed(data.T.reshape(-1), table_t, tail_t)
    return out.reshape(_FIELDS, _EMBED, _BATCH).transpose(2, 0, 1)


# speedup vs baseline: 3.0864x; 1.0070x over previous
import functools

import jax
import jax.numpy as jnp
from jax import lax
from jax.experimental import pallas as pl
from jax.experimental.pallas import tpu as pltpu
from jax.experimental.pallas import tpu_sc as plsc

_FIELDS = 26
_FIELD_DIM = 38462
_EMBED = 32
_BATCH = 16384
_N = _BATCH * _FIELDS
_HALF_B = _BATCH // 2
_VECS = _HALF_B // 16
_SEG_SP = 40960
_SEG_HALF = _SEG_SP // 2
_SEG_COLS = 4096
_SEG_ROWS = 16
_SEG_CHUNKS = 10
_UNROLL = 4
_TBL_ALIGNED = 999936
_TAIL = 1000012 - _TBL_ALIGNED
_TAIL_CUT = _FIELD_DIM - _TAIL


def _seg(f):
    start = f * _FIELD_DIM
    a0 = start & ~127
    ln = -(-((start - a0) + _FIELD_DIM) // 128) * 128
    if a0 + ln > _TBL_ALIGNED:
        a0 -= 128
        ln = _TBL_ALIGNED - a0
    return a0, start - a0, ln


@jax.jit
def _sc_embed(data_flat, table_t, tail_t):
    mesh = plsc.VectorSubcoreMesh(core_axis_name="c", subcore_axis_name="s")

    @functools.partial(
        pl.kernel,
        mesh=mesh,
        compiler_params=pltpu.CompilerParams(
            use_tc_tiling_on_sc=True,
            needs_layout_passes=False,
            internal_scratch_in_bytes=1 << 14,
        ),
        out_type=jax.ShapeDtypeStruct((_N * _EMBED,), jnp.float32),
        scratch_types=[
            pltpu.VMEM_SHARED((16, _SEG_HALF), jnp.float32),
            pltpu.VMEM_SHARED((16, _SEG_HALF), jnp.float32),
            pltpu.VMEM_SHARED((16, 128), jnp.float32),
            pltpu.VMEM((_SEG_ROWS, _SEG_COLS), jnp.float32),
            pltpu.VMEM((1, 128), jnp.float32),
            pltpu.VMEM((_HALF_B,), jnp.int32),
            pltpu.VMEM((_HALF_B,), jnp.float32),
            pltpu.SemaphoreType.DMA,
            pltpu.SemaphoreType.DMA,
        ],
    )
    def k(data_hbm, table_hbm, tail_hbm, out_hbm, spmem_a, spmem_b,
          spmem_tail, seg_av, tail_v, idx_v, out_v, sblk, srow):
        c = lax.axis_index("c")
        s = lax.axis_index("s")
        e = c * 16 + s
        erow = pl.multiple_of(c * 16, 16)
        zeros16 = jnp.zeros((16,), jnp.int32)

        def blk_args(f):
            a0, _, ln = _seg(f)
            return [
                (
                    table_hbm.at[pl.ds(erow, 16), pl.ds(a0, _SEG_HALF)],
                    spmem_a,
                    sblk,
                ),
                (
                    table_hbm.at[
                        pl.ds(erow, 16), pl.ds(a0 + _SEG_HALF, ln - _SEG_HALF)
                    ],
                    spmem_b.at[:, pl.ds(0, ln - _SEG_HALF)],
                    sblk,
                ),
            ]

        def blk_start(f):
            @pl.when(s == 0)
            def _():
                for args in blk_args(f):
                    pltpu.async_copy(*args)

        def blk_wait(f):
            @pl.when(s == 0)
            def _():
                for args in blk_args(f):
                    pltpu.make_async_copy(*args).wait()

        def row_fire(dst):
            dmas = []
            for r in range(_SEG_CHUNKS):
                col = r * _SEG_COLS
                sp = spmem_a if col < _SEG_HALF else spmem_b
                col = col if col < _SEG_HALF else col - _SEG_HALF
                dmas.append(
                    pltpu.async_copy(
                        sp.at[pl.ds(s, 1), pl.ds(col, _SEG_COLS)],
                        dst.at[pl.ds(r, 1), :],
                        srow,
                    )
                )
            return dmas

        @pl.when(s == 0)
        def _():
            pltpu.async_copy(tail_hbm.at[pl.ds(erow, 16), :], spmem_tail, sblk)
        blk_start(0)

        @pl.when(s == 0)
        def _():
            pltpu.make_async_copy(
                tail_hbm.at[pl.ds(erow, 16), :], spmem_tail, sblk
            ).wait()
        blk_wait(0)
        plsc.subcore_barrier()
        pltpu.sync_copy(spmem_tail.at[pl.ds(s, 1), :], tail_v)
        for d in row_fire(seg_av):
            d.wait()
        plsc.subcore_barrier()
        blk_start(1)
        row_pend = []

        for f in range(_FIELDS):
            delta = _seg(f)[1]
            seg = seg_av

            if f < _FIELDS - 1:
                def gather(j, carry):
                    for u in range(_UNROLL):
                        sl = pl.ds(j * 16 * _UNROLL + u * 16, 16)
                        iv = idx_v[sl] + delta
                        out_v[sl] = plsc.load_gather(
                            seg, [lax.shift_right_logical(iv, 12), iv & 4095]
                        )
                    return carry
            else:
                def gather(j, carry):
                    for u in range(_UNROLL):
                        sl = pl.ds(j * 16 * _UNROLL + u * 16, 16)
                        raw = idx_v[sl]
                        iv = jnp.minimum(raw, _TAIL_CUT - 1) + delta
                        main = plsc.load_gather(
                            seg, [lax.shift_right_logical(iv, 12), iv & 4095]
                        )
                        tail = plsc.load_gather(
                            tail_v, [zeros16, jnp.maximum(raw - _TAIL_CUT, 0)]
                        )
                        out_v[sl] = jnp.where(raw < _TAIL_CUT, main, tail)
                    return carry

            for h in range(2):
                pltpu.sync_copy(
                    data_hbm.at[pl.ds(f * _BATCH + h * _HALF_B, _HALF_B)],
                    idx_v,
                )
                lax.fori_loop(0, _VECS // _UNROLL, gather, 0)
                dst = pl.multiple_of(
                    (f * _EMBED + e) * _BATCH + h * _HALF_B, _HALF_B
                )
                pltpu.sync_copy(out_v, out_hbm.at[pl.ds(dst, _HALF_B)])

            if f + 1 < _FIELDS:
                blk_wait(f + 1)
                plsc.subcore_barrier()
                for d in row_fire(seg_av):
                    d.wait()
                plsc.subcore_barrier()
                if f + 2 < _FIELDS:
                    blk_start(f + 2)

    return k(data_flat, table_t, tail_t)


def kernel(data, table):
    table_t = table.T
    tail_t = jnp.pad(table_t[:, _TBL_ALIGNED:], ((0, 0), (0, 128 - _TAIL)))
    out = _sc_embed(data.T.reshape(-1), table_t, tail_t)
    return out.reshape(_FIELDS, _EMBED, _BATCH).transpose(2, 0, 1)


# speedup vs baseline: 4.2012x; 1.3612x over previous
import functools

import jax
import jax.numpy as jnp
from jax import lax
from jax.experimental import pallas as pl
from jax.experimental.pallas import tpu as pltpu
from jax.experimental.pallas import tpu_sc as plsc

_FIELDS = 26
_FIELD_DIM = 38462
_EMBED = 32
_BATCH = 16384
_N = _BATCH * _FIELDS
_QB = _BATCH // 4
_VECS = _QB // 16
_SEG_SP = 40960
_SEG_HALF = _SEG_SP // 2
_SEG_B = 18176
_SEG_COLS = 4096
_SEG_ROWS = 16
_SEG_CHUNKS = 10
_UNROLL = 2
_TBL_ALIGNED = 999936
_TAIL = 1000012 - _TBL_ALIGNED
_TAIL_CUT = _FIELD_DIM - _TAIL


def _seg(f):
    start = f * _FIELD_DIM
    a0 = start & ~127
    ln = -(-((start - a0) + _FIELD_DIM) // 128) * 128
    if a0 + ln > _TBL_ALIGNED:
        a0 -= 128
        ln = _TBL_ALIGNED - a0
    return a0, start - a0, ln


@jax.jit
def _sc_embed(data_flat, table_t, tail_t):
    mesh = plsc.VectorSubcoreMesh(core_axis_name="c", subcore_axis_name="s")

    @functools.partial(
        pl.kernel,
        mesh=mesh,
        compiler_params=pltpu.CompilerParams(
            use_tc_tiling_on_sc=True,
            needs_layout_passes=False,
            internal_scratch_in_bytes=1 << 14,
        ),
        out_type=jax.ShapeDtypeStruct((_N * _EMBED,), jnp.float32),
        scratch_types=[
            pltpu.VMEM_SHARED((16, _SEG_HALF), jnp.float32),
            pltpu.VMEM_SHARED((16, _SEG_B), jnp.float32),
            pltpu.VMEM_SHARED((16, 128), jnp.float32),
            pltpu.VMEM((_SEG_ROWS, _SEG_COLS), jnp.float32),
            pltpu.VMEM((1, 128), jnp.float32),
            pltpu.VMEM((_QB,), jnp.int32),
            pltpu.VMEM((_QB,), jnp.int32),
            pltpu.VMEM((_QB,), jnp.float32),
            pltpu.VMEM((_QB,), jnp.float32),
            pltpu.SemaphoreType.DMA,
            pltpu.SemaphoreType.DMA,
            pltpu.SemaphoreType.DMA,
            pltpu.SemaphoreType.DMA,
        ],
    )
    def k(data_hbm, table_hbm, tail_hbm, out_hbm, spmem_a, spmem_b,
          spmem_tail, seg_av, tail_v, idx_v0, idx_v1, out_v0, out_v1,
          sblk, srow, sidx, sout):
        c = lax.axis_index("c")
        s = lax.axis_index("s")
        e = c * 16 + s
        erow = pl.multiple_of(c * 16, 16)
        zeros16 = jnp.zeros((16,), jnp.int32)

        def blk_args(fv):
            a0 = (fv * _FIELD_DIM) & ~127
            if not isinstance(fv, int):
                a0 = pl.multiple_of(a0, 128)
            return [
                (
                    table_hbm.at[pl.ds(erow, 16), pl.ds(a0, _SEG_HALF)],
                    spmem_a,
                    sblk,
                ),
                (
                    table_hbm.at[
                        pl.ds(erow, 16), pl.ds(a0 + _SEG_HALF, _SEG_B)
                    ],
                    spmem_b,
                    sblk,
                ),
            ]

        def blk_args25():
            a0, _, ln = _seg(25)
            return [
                (
                    table_hbm.at[pl.ds(erow, 16), pl.ds(a0, _SEG_HALF)],
                    spmem_a,
                    sblk,
                ),
                (
                    table_hbm.at[
                        pl.ds(erow, 16), pl.ds(a0 + _SEG_HALF, ln - _SEG_HALF)
                    ],
                    spmem_b.at[:, pl.ds(0, ln - _SEG_HALF)],
                    sblk,
                ),
            ]

        def blk_start(args_list, cond=None):
            @pl.when((s == 0) if cond is None else ((s == 0) & cond))
            def _():
                for args in args_list:
                    pltpu.async_copy(*args)

        def blk_wait(args_list, cond=None):
            @pl.when((s == 0) if cond is None else ((s == 0) & cond))
            def _():
                for args in args_list:
                    pltpu.make_async_copy(*args).wait()

        def row_fire(dst):
            dmas = []
            for r in range(_SEG_CHUNKS):
                col = r * _SEG_COLS
                sp = spmem_a if col < _SEG_HALF else spmem_b
                col = col if col < _SEG_HALF else col - _SEG_HALF
                w = min(_SEG_COLS, (_SEG_B if sp is spmem_b else _SEG_HALF) - col)
                dmas.append(
                    pltpu.async_copy(
                        sp.at[pl.ds(s, 1), pl.ds(col, w)],
                        dst.at[pl.ds(r, 1), pl.ds(0, w)],
                        srow,
                    )
                )
            return dmas

        idxb = (idx_v0, idx_v1)
        outb = (out_v0, out_v1)

        def idx_args(fv, h):
            off = fv * _BATCH + h * _QB
            if not isinstance(fv, int):
                off = pl.multiple_of(off, _QB)
            return (data_hbm.at[pl.ds(off, _QB)], idxb[h % 2], sidx)

        def out_args(fv, h):
            dst = pl.multiple_of((fv * _EMBED + e) * _BATCH + h * _QB, _QB)
            return (outb[h % 2], out_hbm.at[pl.ds(dst, _QB)], sout)

        def mk_gather(b, delta, is_tail):
            if not is_tail:
                def gather(j, carry):
                    for u in range(_UNROLL):
                        sl = pl.ds(j * 16 * _UNROLL + u * 16, 16)
                        iv = idxb[b][sl] + delta
                        outb[b][sl] = plsc.load_gather(
                            seg_av,
                            [lax.shift_right_logical(iv, 12), iv & 4095],
                        )
                    return carry
            else:
                def gather(j, carry):
                    for u in range(_UNROLL):
                        sl = pl.ds(j * 16 * _UNROLL + u * 16, 16)
                        raw = idxb[b][sl]
                        iv = jnp.minimum(raw, _TAIL_CUT - 1) + delta
                        main = plsc.load_gather(
                            seg_av,
                            [lax.shift_right_logical(iv, 12), iv & 4095],
                        )
                        tail = plsc.load_gather(
                            tail_v,
                            [zeros16, jnp.maximum(raw - _TAIL_CUT, 0)],
                        )
                        outb[b][sl] = jnp.where(raw < _TAIL_CUT, main, tail)
                    return carry
            return gather

        def quarters(fv, delta, is_tail, first=False, last=False):
            for h in range(4):
                b = h % 2
                pltpu.make_async_copy(*idx_args(fv, h)).wait()
                if not (first and h < 2):
                    if h < 2:
                        pltpu.make_async_copy(*out_args(fv - 1, h + 2)).wait()
                    else:
                        pltpu.make_async_copy(*out_args(fv, h - 2)).wait()
                lax.fori_loop(
                    0, _VECS // _UNROLL, mk_gather(b, delta, is_tail), 0
                )
                if h < 2:
                    pltpu.async_copy(*idx_args(fv, h + 2))
                elif not last:
                    pltpu.async_copy(*idx_args(fv + 1, h - 2))
                pltpu.async_copy(*out_args(fv, h))

        def row_phase(wait_list, start_list, wait_cond=None, start_cond=None):
            blk_wait(wait_list, wait_cond)
            plsc.subcore_barrier()
            for d in row_fire(seg_av):
                d.wait()
            plsc.subcore_barrier()
            if start_list is not None:
                blk_start(start_list, start_cond)

        @pl.when(s == 0)
        def _():
            pltpu.async_copy(tail_hbm.at[pl.ds(erow, 16), :], spmem_tail, sblk)
        blk_start(blk_args(0))

        @pl.when(s == 0)
        def _():
            pltpu.make_async_copy(
                tail_hbm.at[pl.ds(erow, 16), :], spmem_tail, sblk
            ).wait()
        blk_wait(blk_args(0))
        plsc.subcore_barrier()
        pltpu.sync_copy(spmem_tail.at[pl.ds(s, 1), :], tail_v)
        for d in row_fire(seg_av):
            d.wait()
        plsc.subcore_barrier()
        blk_start(blk_args(1))
        pltpu.async_copy(*idx_args(0, 0))
        pltpu.async_copy(*idx_args(0, 1))

        quarters(0, 0, False, first=True)
        row_phase(blk_args(1), blk_args(2))

        def body(f, carry):
            a0 = pl.multiple_of((f * _FIELD_DIM) & ~127, 128)
            delta = f * _FIELD_DIM - a0
            quarters(f, delta, False)
            blk_wait(blk_args(f + 1), f <= 23)
            blk_wait(blk_args25(), f == 24)
            plsc.subcore_barrier()
            for d in row_fire(seg_av):
                d.wait()
            plsc.subcore_barrier()
            blk_start(blk_args(f + 2), f <= 22)
            blk_start(blk_args25(), f == 23)
            return carry

        lax.fori_loop(1, _FIELDS - 1, body, 0)

        quarters(25, _seg(25)[1], True, last=True)
        pltpu.make_async_copy(*out_args(25, 2)).wait()
        pltpu.make_async_copy(*out_args(25, 3)).wait()

    return k(data_flat, table_t, tail_t)


def kernel(data, table):
    table_t = table.T
    tail_t = jnp.pad(table_t[:, _TBL_ALIGNED:], ((0, 0), (0, 128 - _TAIL)))
    out = _sc_embed(data.T.reshape(-1), table_t, tail_t)
    return out.reshape(_FIELDS, _EMBED, _BATCH).transpose(2, 0, 1)


# speedup vs baseline: 4.2147x; 1.0032x over previous
import functools

import jax
import jax.numpy as jnp
from jax import lax
from jax.experimental import pallas as pl
from jax.experimental.pallas import tpu as pltpu
from jax.experimental.pallas import tpu_sc as plsc

_FIELDS = 26
_FIELD_DIM = 38462
_EMBED = 32
_BATCH = 16384
_N = _BATCH * _FIELDS
_QB = _BATCH // 4
_VECS = _QB // 16
_SEG_SP = 40960
_SEG_HALF = _SEG_SP // 2
_SEG_B = 18176
_SEG_COLS = 4096
_SEG_ROWS = 16
_SEG_CHUNKS = 10
_UNROLL = 8
_TBL_ALIGNED = 999936
_TAIL = 1000012 - _TBL_ALIGNED
_TAIL_CUT = _FIELD_DIM - _TAIL


def _seg(f):
    start = f * _FIELD_DIM
    a0 = start & ~127
    ln = -(-((start - a0) + _FIELD_DIM) // 128) * 128
    if a0 + ln > _TBL_ALIGNED:
        a0 -= 128
        ln = _TBL_ALIGNED - a0
    return a0, start - a0, ln


@jax.jit
def _sc_embed(data_flat, table_t, tail_t):
    mesh = plsc.VectorSubcoreMesh(core_axis_name="c", subcore_axis_name="s")

    @functools.partial(
        pl.kernel,
        mesh=mesh,
        compiler_params=pltpu.CompilerParams(
            use_tc_tiling_on_sc=True,
            needs_layout_passes=False,
            internal_scratch_in_bytes=1 << 14,
        ),
        out_type=jax.ShapeDtypeStruct((_N * _EMBED,), jnp.float32),
        scratch_types=[
            pltpu.VMEM_SHARED((16, _SEG_HALF), jnp.float32),
            pltpu.VMEM_SHARED((16, _SEG_B), jnp.float32),
            pltpu.VMEM_SHARED((16, 128), jnp.float32),
            pltpu.VMEM((_SEG_ROWS, _SEG_COLS), jnp.float32),
            pltpu.VMEM((1, 128), jnp.float32),
            pltpu.VMEM((_QB,), jnp.int32),
            pltpu.VMEM((_QB,), jnp.int32),
            pltpu.VMEM((_QB,), jnp.float32),
            pltpu.VMEM((_QB,), jnp.float32),
            pltpu.SemaphoreType.DMA,
            pltpu.SemaphoreType.DMA,
            pltpu.SemaphoreType.DMA,
            pltpu.SemaphoreType.DMA,
        ],
    )
    def k(data_hbm, table_hbm, tail_hbm, out_hbm, spmem_a, spmem_b,
          spmem_tail, seg_av, tail_v, idx_v0, idx_v1, out_v0, out_v1,
          sblk, srow, sidx, sout):
        c = lax.axis_index("c")
        s = lax.axis_index("s")
        e = c * 16 + s
        erow = pl.multiple_of(c * 16, 16)
        zeros16 = jnp.zeros((16,), jnp.int32)

        def blk_args(fv):
            a0 = (fv * _FIELD_DIM) & ~127
            if not isinstance(fv, int):
                a0 = pl.multiple_of(a0, 128)
            return [
                (
                    table_hbm.at[pl.ds(erow, 16), pl.ds(a0, _SEG_HALF)],
                    spmem_a,
                    sblk,
                ),
                (
                    table_hbm.at[
                        pl.ds(erow, 16), pl.ds(a0 + _SEG_HALF, _SEG_B)
                    ],
                    spmem_b,
                    sblk,
                ),
            ]

        def blk_args25():
            a0, _, ln = _seg(25)
            return [
                (
                    table_hbm.at[pl.ds(erow, 16), pl.ds(a0, _SEG_HALF)],
                    spmem_a,
                    sblk,
                ),
                (
                    table_hbm.at[
                        pl.ds(erow, 16), pl.ds(a0 + _SEG_HALF, ln - _SEG_HALF)
                    ],
                    spmem_b.at[:, pl.ds(0, ln - _SEG_HALF)],
                    sblk,
                ),
            ]

        def blk_start(args_list, cond=None):
            @pl.when((s == 0) if cond is None else ((s == 0) & cond))
            def _():
                for args in args_list:
                    pltpu.async_copy(*args)

        def blk_wait(args_list, cond=None):
            @pl.when((s == 0) if cond is None else ((s == 0) & cond))
            def _():
                for args in args_list:
                    pltpu.make_async_copy(*args).wait()

        def row_fire(dst):
            dmas = []
            for r in range(_SEG_CHUNKS):
                col = r * _SEG_COLS
                sp = spmem_a if col < _SEG_HALF else spmem_b
                col = col if col < _SEG_HALF else col - _SEG_HALF
                w = min(_SEG_COLS, (_SEG_B if sp is spmem_b else _SEG_HALF) - col)
                dmas.append(
                    pltpu.async_copy(
                        sp.at[pl.ds(s, 1), pl.ds(col, w)],
                        dst.at[pl.ds(r, 1), pl.ds(0, w)],
                        srow,
                    )
                )
            return dmas

        idxb = (idx_v0, idx_v1)
        outb = (out_v0, out_v1)

        def idx_args(fv, h):
            off = fv * _BATCH + h * _QB
            if not isinstance(fv, int):
                off = pl.multiple_of(off, _QB)
            return (data_hbm.at[pl.ds(off, _QB)], idxb[h % 2], sidx)

        def out_args(fv, h):
            dst = pl.multiple_of((fv * _EMBED + e) * _BATCH + h * _QB, _QB)
            return (outb[h % 2], out_hbm.at[pl.ds(dst, _QB)], sout)

        def mk_gather(b, delta, is_tail):
            if not is_tail:
                def gather(j, carry):
                    for u in range(_UNROLL):
                        sl = pl.ds(j * 16 * _UNROLL + u * 16, 16)
                        iv = idxb[b][sl] + delta
                        outb[b][sl] = plsc.load_gather(
                            seg_av,
                            [lax.shift_right_logical(iv, 12), iv & 4095],
                        )
                    return carry
            else:
                def gather(j, carry):
                    for u in range(_UNROLL):
                        sl = pl.ds(j * 16 * _UNROLL + u * 16, 16)
                        raw = idxb[b][sl]
                        iv = jnp.minimum(raw, _TAIL_CUT - 1) + delta
                        main = plsc.load_gather(
                            seg_av,
                            [lax.shift_right_logical(iv, 12), iv & 4095],
                        )
                        tail = plsc.load_gather(
                            tail_v,
                            [zeros16, jnp.maximum(raw - _TAIL_CUT, 0)],
                        )
                        outb[b][sl] = jnp.where(raw < _TAIL_CUT, main, tail)
                    return carry
            return gather

        def quarters(fv, delta, is_tail, first=False, last=False):
            for h in range(4):
                b = h % 2
                pltpu.make_async_copy(*idx_args(fv, h)).wait()
                if not (first and h < 2):
                    if h < 2:
                        pltpu.make_async_copy(*out_args(fv - 1, h + 2)).wait()
                    else:
                        pltpu.make_async_copy(*out_args(fv, h - 2)).wait()
                lax.fori_loop(
                    0, _VECS // _UNROLL, mk_gather(b, delta, is_tail), 0
                )
                if h < 2:
                    pltpu.async_copy(*idx_args(fv, h + 2))
                elif not last:
                    pltpu.async_copy(*idx_args(fv + 1, h - 2))
                pltpu.async_copy(*out_args(fv, h))

        def row_phase(wait_list, start_list, wait_cond=None, start_cond=None):
            blk_wait(wait_list, wait_cond)
            plsc.subcore_barrier()
            for d in row_fire(seg_av):
                d.wait()
            plsc.subcore_barrier()
            if start_list is not None:
                blk_start(start_list, start_cond)

        @pl.when(s == 0)
        def _():
            pltpu.async_copy(tail_hbm.at[pl.ds(erow, 16), :], spmem_tail, sblk)
        blk_start(blk_args(0))

        @pl.when(s == 0)
        def _():
            pltpu.make_async_copy(
                tail_hbm.at[pl.ds(erow, 16), :], spmem_tail, sblk
            ).wait()
        blk_wait(blk_args(0))
        plsc.subcore_barrier()
        pltpu.sync_copy(spmem_tail.at[pl.ds(s, 1), :], tail_v)
        for d in row_fire(seg_av):
            d.wait()
        plsc.subcore_barrier()
        blk_start(blk_args(1))
        pltpu.async_copy(*idx_args(0, 0))
        pltpu.async_copy(*idx_args(0, 1))

        quarters(0, 0, False, first=True)
        row_phase(blk_args(1), blk_args(2))

        def body(f, carry):
            a0 = pl.multiple_of((f * _FIELD_DIM) & ~127, 128)
            delta = f * _FIELD_DIM - a0
            quarters(f, delta, False)
            blk_wait(blk_args(f + 1), f <= 23)
            blk_wait(blk_args25(), f == 24)
            plsc.subcore_barrier()
            for d in row_fire(seg_av):
                d.wait()
            plsc.subcore_barrier()
            blk_start(blk_args(f + 2), f <= 22)
            blk_start(blk_args25(), f == 23)
            return carry

        lax.fori_loop(1, _FIELDS - 1, body, 0)

        quarters(25, _seg(25)[1], True, last=True)
        pltpu.make_async_copy(*out_args(25, 2)).wait()
        pltpu.make_async_copy(*out_args(25, 3)).wait()

    return k(data_flat, table_t, tail_t)


def kernel(data, table):
    table_t = table.T
    tail_t = jnp.pad(table_t[:, _TBL_ALIGNED:], ((0, 0), (0, 128 - _TAIL)))
    out = _sc_embed(data.T.reshape(-1), table_t, tail_t)
    return out.reshape(_FIELDS, _EMBED, _BATCH).transpose(2, 0, 1)
